# Initial kernel scaffold; baseline (speedup 1.0000x reference)
import jax
import jax.numpy as jnp
from jax.experimental import pallas as pl


def kernel(x, edge_index, Wp0, bp0, Ws0, Wn0, b0, Wp1, bp1, Ws1, Wn1, b1, Wp2, bp2, Ws2, Wn2, b2, Wp3, bp3, Ws3, Wn3, b3):
    raise NotImplementedError("write your pallas kernel here")


# speedup vs baseline: 5.0528x; 5.0528x over previous
import jax
import jax.numpy as jnp
from jax import lax
from jax.experimental import pallas as pl
from jax.experimental.pallas import tpu as pltpu
from jax.experimental.pallas import tpu_sc as plsc

N = 10000
E = 320000
NC = 2
NS = 16
W = NC * NS
R = 320
NPAD = W * R
CAP = 16384
G = 128
CAPL = CAP // 16
CAPP = CAP + G
CAPJ = CAPP + 16
CE = 6400
RH = R + 1
HSZ = 16 * RH + 16
HSZP = ((HSZ + 15) // 16) * 16
CSZ = 16 * R + 16

_mesh = plsc.VectorSubcoreMesh(core_axis_name="c", subcore_axis_name="s")


def _worker_id():
    return lax.axis_index("s") * NC + lax.axis_index("c")


def _prep_body(src_hbm, dst_hbm, srcs_hbm, dls_hbm, cnt_hbm,
               sbuf, dbuf, srcl_v, dstl_v, ssort_v, dsort_v,
               hist_v, cur2_v, ptmp, cntv):
    w = _worker_id()
    lo = w * R
    iota = lax.iota(jnp.int32, 16)
    lane_base = iota * CAPL
    zero16 = jnp.zeros((16,), jnp.int32)

    def chunk(ci, cur_v):
        pltpu.sync_copy(src_hbm.at[pl.ds(ci * CE, CE)], sbuf)
        pltpu.sync_copy(dst_hbm.at[pl.ds(ci * CE, CE)], dbuf)

        def vec(j, cur_v):
            dv = dbuf[pl.ds(j * 16, 16)]
            sv = sbuf[pl.ds(j * 16, 16)]
            dl = dv - lo
            m = (dl >= 0) & (dl < R) & (cur_v < CAPL)
            pos = jnp.where(m, lane_base + cur_v, CAPP + iota)
            plsc.store_scatter(srcl_v, [pos], sv)
            plsc.store_scatter(dstl_v, [pos], dl)
            return cur_v + m.astype(jnp.int32)

        return lax.fori_loop(0, CE // 16, vec, cur_v)

    cur_v = lax.fori_loop(0, E // CE, chunk, zero16)

    maxc = cur_v[0]
    for l in range(1, 16):
        maxc = jnp.maximum(maxc, cur_v[l])

    def zh(i, _):
        hist_v[pl.ds(i * 16, 16)] = zero16
        return 0

    lax.fori_loop(0, HSZP // 16, zh, 0)

    def hstep(i, _):
        dlv = plsc.load_gather(dstl_v, [lane_base + i])
        msk = i < cur_v
        pos = jnp.where(msk, iota * RH + dlv, 16 * RH + iota)
        h = plsc.load_gather(hist_v, [pos])
        plsc.store_scatter(hist_v, [pos], h + 1)
        return 0

    lax.fori_loop(0, maxc, hstep, 0)

    ptmp[pl.ds(0, 16)] = zero16
    run_base = jnp.int32(0)
    for nb in range(R // 16):
        part = zero16
        parts = []
        for l in range(16):
            parts.append(part)
            part = part + hist_v[pl.ds(l * RH + nb * 16, 16)]
        tot = part
        x = tot
        for k in (1, 2, 4, 8):
            ptmp[pl.ds(16, 16)] = x
            x = x + ptmp[pl.ds(16 - k, 16)]
        base_vec = run_base + (x - tot)
        run_base = run_base + x[15]
        npos = (nb * 16 + iota) * 16
        for l in range(16):
            plsc.store_scatter(cur2_v, [npos + l], base_vec + parts[l])

    cnt = run_base

    def pstep(i, _):
        dlv = plsc.load_gather(dstl_v, [lane_base + i])
        srv = plsc.load_gather(srcl_v, [lane_base + i])
        msk = i < cur_v
        cpos = jnp.where(msk, dlv * 16 + iota, 16 * R + iota)
        pv = plsc.load_gather(cur2_v, [cpos])
        plsc.store_scatter(cur2_v, [cpos], pv + 1)
        opos = jnp.where(msk, pv, CAPP + iota)
        plsc.store_scatter(ssort_v, [opos], srv)
        plsc.store_scatter(dsort_v, [opos], dlv)
        return 0

    lax.fori_loop(0, maxc, pstep, 0)

    cnt_pad = ((cnt + G - 1) // G) * G
    for k in range(G // 16):
        pos = cnt + k * 16 + iota
        pos = jnp.where(pos < cnt_pad, pos, CAPP + iota)
        plsc.store_scatter(ssort_v, [pos], lo + iota)
        plsc.store_scatter(dsort_v, [pos], jnp.full((16,), R, jnp.int32))
    for k in range(8):
        cntv[pl.ds(k * 16, 16)] = jnp.broadcast_to(cnt_pad, (16,)).astype(
            jnp.int32)
    pltpu.sync_copy(cntv, cnt_hbm.at[pl.ds(w * 128, 128)])
    pltpu.sync_copy(ssort_v.at[pl.ds(0, CAPP)],
                    srcs_hbm.at[pl.ds(w * CAPP, CAPP)])
    pltpu.sync_copy(dsort_v.at[pl.ds(0, CAPP)],
                    dls_hbm.at[pl.ds(w * CAPP, CAPP)])


_prep = pl.kernel(
    _prep_body,
    out_type=(
        jax.ShapeDtypeStruct((W * CAPP,), jnp.int32),
        jax.ShapeDtypeStruct((W * CAPP,), jnp.int32),
        jax.ShapeDtypeStruct((W * 128,), jnp.int32),
    ),
    mesh=_mesh,
    compiler_params=pltpu.CompilerParams(needs_layout_passes=False),
    scratch_types=[
        pltpu.VMEM((CE,), jnp.int32),
        pltpu.VMEM((CE,), jnp.int32),
        pltpu.VMEM((CAPJ,), jnp.int32),
        pltpu.VMEM((CAPJ,), jnp.int32),
        pltpu.VMEM((CAPJ,), jnp.int32),
        pltpu.VMEM((CAPJ,), jnp.int32),
        pltpu.VMEM((HSZP,), jnp.int32),
        pltpu.VMEM((CSZ,), jnp.int32),
        pltpu.VMEM((32,), jnp.int32),
        pltpu.VMEM((128,), jnp.int32),
    ],
)


D = 128
NSL = D // 16


def _segmax_body(p_hbm, srcl_hbm, dstl_hbm, cnt_hbm, out_hbm,
                 idx0, idx1, dl0, dl1, rows0, rows1, acc, cntv, sem0, sem1):
    w = _worker_id()
    lo = w * R
    zero = jnp.zeros((16,), jnp.float32)
    idx = (idx0, idx1)
    dl = (dl0, dl1)
    rows = (rows0, rows1)
    sem = (sem0, sem1)

    def zrow(i, _):
        for s_ in range(NSL):
            acc[i, pl.ds(s_ * 16, 16)] = zero
        return 0

    lax.fori_loop(0, R + 1, zrow, 0)

    pltpu.sync_copy(cnt_hbm.at[pl.ds(w * 128, 128)], cntv)
    cnt_pad = cntv[pl.ds(0, 16)][0]
    trips = cnt_pad // G

    def fetch(g, b):
        base = w * CAPP + g * G
        pltpu.sync_copy(srcl_hbm.at[pl.ds(base, G)], idx[b])
        pltpu.sync_copy(dstl_hbm.at[pl.ds(base, G)], dl[b])
        pltpu.async_copy(p_hbm.at[idx[b]], rows[b], sem[b])

    def process(b):
        def group(jg, _):
            dvec = dl[b][pl.ds(jg * 16, 16)]
            d_prev = dvec[0]
            regs = [acc[d_prev, pl.ds(s_ * 16, 16)] for s_ in range(NSL)]
            for j2 in range(16):
                row = jg * 16 + j2
                rv = [rows[b][row, pl.ds(s_ * 16, 16)] for s_ in range(NSL)]
                if j2 == 0:
                    regs = [jnp.maximum(a_, r_) for a_, r_ in zip(regs, rv)]
                else:
                    d = dvec[j2]
                    same = d == d_prev
                    dp = d_prev

                    @pl.when(jnp.logical_not(same))
                    def _(regs=tuple(regs), dp=dp):
                        for s_ in range(NSL):
                            acc[dp, pl.ds(s_ * 16, 16)] = regs[s_]

                    regs = [jnp.where(same, jnp.maximum(a_, r_), r_)
                            for a_, r_ in zip(regs, rv)]
                    d_prev = d
            for s_ in range(NSL):
                acc[d_prev, pl.ds(s_ * 16, 16)] = regs[s_]
            return 0

        lax.fori_loop(0, G // 16, group, 0)

    @pl.when(trips > 0)
    def _():
        fetch(0, 0)

    def pair(g2, _):
        for b in range(2):
            g = g2 * 2 + b

            @pl.when(g < trips)
            def _():
                @pl.when(g + 1 < trips)
                def _():
                    fetch(g + 1, 1 - b)

                pltpu.make_async_copy(p_hbm.at[idx[b]], rows[b],
                                      sem[b]).wait()
                process(b)
        return 0

    lax.fori_loop(0, (trips + 1) // 2, pair, 0)
    pltpu.sync_copy(acc.at[pl.ds(0, R)], out_hbm.at[pl.ds(lo, R)])


_segmax = pl.kernel(
    _segmax_body,
    out_type=jax.ShapeDtypeStruct((NPAD, D), jnp.float32),
    mesh=_mesh,
    compiler_params=pltpu.CompilerParams(needs_layout_passes=False),
    scratch_types=[
        pltpu.VMEM((G,), jnp.int32),
        pltpu.VMEM((G,), jnp.int32),
        pltpu.VMEM((G,), jnp.int32),
        pltpu.VMEM((G,), jnp.int32),
        pltpu.VMEM((G, D), jnp.float32),
        pltpu.VMEM((G, D), jnp.float32),
        pltpu.VMEM((R + 1, D), jnp.float32),
        pltpu.VMEM((128,), jnp.int32),
        pltpu.SemaphoreType.DMA,
        pltpu.SemaphoreType.DMA,
    ],
)


_BM = 2560


def _linear_relu_body(h_ref, w_ref, b_ref, o_ref):
    o_ref[...] = jnp.maximum(
        jnp.dot(h_ref[...], w_ref[...],
                preferred_element_type=jnp.float32,
                precision=lax.Precision.HIGHEST) + b_ref[...], 0.0)


def _linear_relu(h, w_, b_):
    k, d = w_.shape
    return pl.pallas_call(
        _linear_relu_body,
        grid=(NPAD // _BM,),
        in_specs=[
            pl.BlockSpec((_BM, k), lambda i: (i, 0)),
            pl.BlockSpec((k, d), lambda i: (0, 0)),
            pl.BlockSpec((1, d), lambda i: (0, 0)),
        ],
        out_specs=pl.BlockSpec((_BM, d), lambda i: (i, 0)),
        out_shape=jax.ShapeDtypeStruct((NPAD, d), jnp.float32),
    )(h, w_, b_.reshape(1, d))


def _out_relu_body(h_ref, ws_ref, hn_ref, wn_ref, b_ref, o_ref):
    acc = jnp.dot(h_ref[...], ws_ref[...],
                  preferred_element_type=jnp.float32,
                  precision=lax.Precision.HIGHEST)
    acc += jnp.dot(hn_ref[...], wn_ref[...],
                   preferred_element_type=jnp.float32,
                   precision=lax.Precision.HIGHEST)
    o_ref[...] = jnp.maximum(acc + b_ref[...], 0.0)


def _out_relu(h, ws, hn, wn, b_):
    k, d = ws.shape
    return pl.pallas_call(
        _out_relu_body,
        grid=(NPAD // _BM,),
        in_specs=[
            pl.BlockSpec((_BM, k), lambda i: (i, 0)),
            pl.BlockSpec((k, d), lambda i: (0, 0)),
            pl.BlockSpec((_BM, k), lambda i: (i, 0)),
            pl.BlockSpec((k, d), lambda i: (0, 0)),
            pl.BlockSpec((1, d), lambda i: (0, 0)),
        ],
        out_specs=pl.BlockSpec((_BM, d), lambda i: (i, 0)),
        out_shape=jax.ShapeDtypeStruct((NPAD, d), jnp.float32),
    )(h, ws, hn, wn, b_.reshape(1, d))


def _pad2(a, r, c):
    return jnp.pad(a, ((0, r - a.shape[0]), (0, c - a.shape[1])))


def _pad1(a, c):
    return jnp.pad(a, ((0, c - a.shape[0]),))


def kernel(x, edge_index,
           Wp0, bp0, Ws0, Wn0, b0,
           Wp1, bp1, Ws1, Wn1, b1,
           Wp2, bp2, Ws2, Wn2, b2,
           Wp3, bp3, Ws3, Wn3, b3):
    src = edge_index[0].astype(jnp.int32)
    dst = edge_index[1].astype(jnp.int32)
    srcl, dstl, cnt = _prep(src, dst)

    h = jnp.pad(x, ((0, NPAD - N), (0, 0)))
    params = [(Wp0, bp0, Ws0, Wn0, b0), (Wp1, bp1, Ws1, Wn1, b1),
              (Wp2, bp2, Ws2, Wn2, b2), (Wp3, bp3, Ws3, Wn3, b3)]
    for (Wp, bp, Ws, Wn, b) in params:
        p = _linear_relu(h, _pad2(Wp, D, D), _pad1(bp, D))
        hn = _segmax(p, srcl, dstl, cnt)
        h = _out_relu(h, _pad2(Ws, D, D), hn, _pad2(Wn, D, D), _pad1(b, D))
    return h[:N]


# speedup vs baseline: 7.2609x; 1.4370x over previous
import jax
import jax.numpy as jnp
from jax import lax
from jax.experimental import pallas as pl
from jax.experimental.pallas import tpu as pltpu
from jax.experimental.pallas import tpu_sc as plsc

N = 10000
E = 320000
NC = 2
NS = 16
W = NC * NS
R = 320
NPAD = W * R
CAP = 16384
G = 128
CAPL = CAP // 16
CAPP = CAP + G
CAPJ = CAPP + 16
CE = 6400
RH = R + 1
HSZ = 16 * RH + 16
HSZP = ((HSZ + 15) // 16) * 16
CSZ = 16 * R + 16

_mesh = plsc.VectorSubcoreMesh(core_axis_name="c", subcore_axis_name="s")


def _worker_id():
    return lax.axis_index("s") * NC + lax.axis_index("c")


def _prep_body(src_hbm, dst_hbm, srcs_hbm, dls_hbm, cnt_hbm,
               sbuf0, sbuf1, dbuf0, dbuf1, srcl_v, dstl_v, ssort_v, dsort_v,
               hist_v, cur2_v, ptmp, cntv, sema0, sema1, semb0, semb1):
    w = _worker_id()
    lo = w * R
    iota = lax.iota(jnp.int32, 16)
    lane_base = iota * CAPL
    zero16 = jnp.zeros((16,), jnp.int32)

    sbuf = (sbuf0, sbuf1)
    dbuf = (dbuf0, dbuf1)
    sema = (sema0, sema1)
    semb = (semb0, semb1)
    NCHUNK = E // CE

    def fetchA(ci, b):
        pltpu.async_copy(src_hbm.at[pl.ds(ci * CE, CE)], sbuf[b], sema[b])
        pltpu.async_copy(dst_hbm.at[pl.ds(ci * CE, CE)], dbuf[b], semb[b])

    fetchA(0, 0)

    def pairA(ci2, cur_v):
        for b in range(2):
            ci = ci2 * 2 + b

            @pl.when(ci + 1 < NCHUNK)
            def _():
                fetchA(ci + 1, 1 - b)

            pltpu.make_async_copy(src_hbm.at[pl.ds(0, CE)], sbuf[b],
                                  sema[b]).wait()
            pltpu.make_async_copy(dst_hbm.at[pl.ds(0, CE)], dbuf[b],
                                  semb[b]).wait()

            def vec(j, cur_v):
                def one(off, cur_v):
                    dv = dbuf[b][pl.ds(off, 16)]
                    sv = sbuf[b][pl.ds(off, 16)]
                    dl = dv - lo
                    m = (dl >= 0) & (dl < R) & (cur_v < CAPL)
                    pos = jnp.where(m, lane_base + cur_v, CAPP + iota)
                    plsc.store_scatter(srcl_v, [pos], sv)
                    plsc.store_scatter(dstl_v, [pos], dl)
                    return cur_v + m.astype(jnp.int32)

                cur_v = one(j * 32, cur_v)
                return one(j * 32 + 16, cur_v)

            cur_v = lax.fori_loop(0, CE // 32, vec, cur_v)
        return cur_v

    cur_v = lax.fori_loop(0, NCHUNK // 2, pairA, zero16)

    maxc = cur_v[0]
    for l in range(1, 16):
        maxc = jnp.maximum(maxc, cur_v[l])

    def zh(i, _):
        hist_v[pl.ds(i * 16, 16)] = zero16
        return 0

    lax.fori_loop(0, HSZP // 16, zh, 0)

    def hstep(i, _):
        dlv = plsc.load_gather(dstl_v, [lane_base + i])
        msk = i < cur_v
        pos = jnp.where(msk, iota * RH + dlv, 16 * RH + iota)
        h = plsc.load_gather(hist_v, [pos])
        plsc.store_scatter(hist_v, [pos], h + 1)
        return 0

    lax.fori_loop(0, maxc, hstep, 0)

    ptmp[pl.ds(0, 16)] = zero16
    run_base = jnp.int32(0)
    for nb in range(R // 16):
        part = zero16
        parts = []
        for l in range(16):
            parts.append(part)
            part = part + hist_v[pl.ds(l * RH + nb * 16, 16)]
        tot = part
        x = tot
        for k in (1, 2, 4, 8):
            ptmp[pl.ds(16, 16)] = x
            x = x + ptmp[pl.ds(16 - k, 16)]
        base_vec = run_base + (x - tot)
        run_base = run_base + x[15]
        npos = (nb * 16 + iota) * 16
        for l in range(16):
            plsc.store_scatter(cur2_v, [npos + l], base_vec + parts[l])

    cnt = run_base

    def pstep(i, _):
        dlv = plsc.load_gather(dstl_v, [lane_base + i])
        srv = plsc.load_gather(srcl_v, [lane_base + i])
        msk = i < cur_v
        cpos = jnp.where(msk, dlv * 16 + iota, 16 * R + iota)
        pv = plsc.load_gather(cur2_v, [cpos])
        plsc.store_scatter(cur2_v, [cpos], pv + 1)
        opos = jnp.where(msk, pv, CAPP + iota)
        plsc.store_scatter(ssort_v, [opos], srv)
        plsc.store_scatter(dsort_v, [opos], dlv)
        return 0

    lax.fori_loop(0, maxc, pstep, 0)

    cnt_pad = ((cnt + G - 1) // G) * G
    for k in range(G // 16):
        pos = cnt + k * 16 + iota
        pos = jnp.where(pos < cnt_pad, pos, CAPP + iota)
        plsc.store_scatter(ssort_v, [pos], lo + iota)
        plsc.store_scatter(dsort_v, [pos], jnp.full((16,), R, jnp.int32))
    for k in range(8):
        cntv[pl.ds(k * 16, 16)] = jnp.broadcast_to(cnt_pad, (16,)).astype(
            jnp.int32)
    pltpu.sync_copy(cntv, cnt_hbm.at[pl.ds(w * 128, 128)])
    pltpu.sync_copy(ssort_v.at[pl.ds(0, CAPP)],
                    srcs_hbm.at[pl.ds(w * CAPP, CAPP)])
    pltpu.sync_copy(dsort_v.at[pl.ds(0, CAPP)],
                    dls_hbm.at[pl.ds(w * CAPP, CAPP)])


_prep = pl.kernel(
    _prep_body,
    out_type=(
        jax.ShapeDtypeStruct((W * CAPP,), jnp.int32),
        jax.ShapeDtypeStruct((W * CAPP,), jnp.int32),
        jax.ShapeDtypeStruct((W * 128,), jnp.int32),
    ),
    mesh=_mesh,
    compiler_params=pltpu.CompilerParams(needs_layout_passes=False),
    scratch_types=[
        pltpu.VMEM((CE,), jnp.int32),
        pltpu.VMEM((CE,), jnp.int32),
        pltpu.VMEM((CE,), jnp.int32),
        pltpu.VMEM((CE,), jnp.int32),
        pltpu.VMEM((CAPJ,), jnp.int32),
        pltpu.VMEM((CAPJ,), jnp.int32),
        pltpu.VMEM((CAPJ,), jnp.int32),
        pltpu.VMEM((CAPJ,), jnp.int32),
        pltpu.VMEM((HSZP,), jnp.int32),
        pltpu.VMEM((CSZ,), jnp.int32),
        pltpu.VMEM((32,), jnp.int32),
        pltpu.VMEM((128,), jnp.int32),
        pltpu.SemaphoreType.DMA,
        pltpu.SemaphoreType.DMA,
        pltpu.SemaphoreType.DMA,
        pltpu.SemaphoreType.DMA,
    ],
)


D = 128
NSL = D // 16


def _segmax_body(p_hbm, srcl_hbm, dstl_hbm, cnt_hbm, out_hbm,
                 idx0, idx1, idx2, dl0, dl1, dl2, rows0, rows1, rows2,
                 acc, cntv, sem0, sem1, sem2, semi0, semi1, semi2):
    w = _worker_id()
    lo = w * R
    zero = jnp.zeros((16,), jnp.float32)
    idx = (idx0, idx1, idx2)
    dl = (dl0, dl1, dl2)
    rows = (rows0, rows1, rows2)
    sem = (sem0, sem1, sem2)
    semi = (semi0, semi1, semi2)

    pltpu.sync_copy(cnt_hbm.at[pl.ds(w * 128, 128)], cntv)
    cnt_pad = cntv[pl.ds(0, 16)][0]
    trips = cnt_pad // G

    def load_idx(g, b):
        base = w * CAPP + g * G
        pltpu.async_copy(srcl_hbm.at[pl.ds(base, G)], idx[b], semi[b])
        pltpu.async_copy(dstl_hbm.at[pl.ds(base, G)], dl[b], semi[b])

    def start_gather(b):
        pltpu.make_async_copy(srcl_hbm.at[pl.ds(0, G)], idx[b],
                              semi[b]).wait()
        pltpu.make_async_copy(dstl_hbm.at[pl.ds(0, G)], dl[b],
                              semi[b]).wait()
        pltpu.async_copy(p_hbm.at[idx[b]], rows[b], sem[b])

    @pl.when(trips > 0)
    def _():
        load_idx(0, 0)

    @pl.when(trips > 1)
    def _():
        load_idx(1, 1)

    @pl.when(trips > 0)
    def _():
        start_gather(0)

    def zrow(i, _):
        for s_ in range(NSL):
            acc[i, pl.ds(s_ * 16, 16)] = zero
        return 0

    lax.fori_loop(0, R + 1, zrow, 0)

    def process(b):
        def group(jg, _):
            dvec = dl[b][pl.ds(jg * 16, 16)]
            d_prev = dvec[0]
            regs = [acc[d_prev, pl.ds(s_ * 16, 16)] for s_ in range(NSL)]
            for j2 in range(16):
                row = jg * 16 + j2
                rv = [rows[b][row, pl.ds(s_ * 16, 16)] for s_ in range(NSL)]
                if j2 == 0:
                    regs = [jnp.maximum(a_, r_) for a_, r_ in zip(regs, rv)]
                else:
                    d = dvec[j2]
                    same = d == d_prev
                    dp = d_prev

                    @pl.when(jnp.logical_not(same))
                    def _(regs=tuple(regs), dp=dp):
                        for s_ in range(NSL):
                            acc[dp, pl.ds(s_ * 16, 16)] = regs[s_]

                    regs = [jnp.where(same, jnp.maximum(a_, r_), r_)
                            for a_, r_ in zip(regs, rv)]
                    d_prev = d
            for s_ in range(NSL):
                acc[d_prev, pl.ds(s_ * 16, 16)] = regs[s_]
            return 0

        lax.fori_loop(0, G // 16, group, 0)

    def tri(g3, _):
        for b in range(3):
            g = g3 * 3 + b

            @pl.when(g < trips)
            def _():
                @pl.when(g + 2 < trips)
                def _():
                    load_idx(g + 2, (b + 2) % 3)

                @pl.when(g + 1 < trips)
                def _():
                    start_gather((b + 1) % 3)

                pltpu.make_async_copy(p_hbm.at[idx[b]], rows[b],
                                      sem[b]).wait()
                process(b)
        return 0

    lax.fori_loop(0, (trips + 2) // 3, tri, 0)
    pltpu.sync_copy(acc.at[pl.ds(0, R)], out_hbm.at[pl.ds(lo, R)])


_segmax = pl.kernel(
    _segmax_body,
    out_type=jax.ShapeDtypeStruct((NPAD, D), jnp.float32),
    mesh=_mesh,
    compiler_params=pltpu.CompilerParams(needs_layout_passes=False),
    scratch_types=[
        pltpu.VMEM((G,), jnp.int32),
        pltpu.VMEM((G,), jnp.int32),
        pltpu.VMEM((G,), jnp.int32),
        pltpu.VMEM((G,), jnp.int32),
        pltpu.VMEM((G,), jnp.int32),
        pltpu.VMEM((G,), jnp.int32),
        pltpu.VMEM((G, D), jnp.float32),
        pltpu.VMEM((G, D), jnp.float32),
        pltpu.VMEM((G, D), jnp.float32),
        pltpu.VMEM((R + 1, D), jnp.float32),
        pltpu.VMEM((128,), jnp.int32),
        pltpu.SemaphoreType.DMA,
        pltpu.SemaphoreType.DMA,
        pltpu.SemaphoreType.DMA,
        pltpu.SemaphoreType.DMA,
        pltpu.SemaphoreType.DMA,
        pltpu.SemaphoreType.DMA,
    ],
)


_BM = 2560


def _linear_relu_body(h_ref, w_ref, b_ref, o_ref):
    o_ref[...] = jnp.maximum(
        jnp.dot(h_ref[...], w_ref[...],
                preferred_element_type=jnp.float32,
                precision=lax.Precision.HIGHEST) + b_ref[...], 0.0)


def _linear_relu(h, w_, b_):
    k, d = w_.shape
    return pl.pallas_call(
        _linear_relu_body,
        grid=(NPAD // _BM,),
        in_specs=[
            pl.BlockSpec((_BM, k), lambda i: (i, 0)),
            pl.BlockSpec((k, d), lambda i: (0, 0)),
            pl.BlockSpec((1, d), lambda i: (0, 0)),
        ],
        out_specs=pl.BlockSpec((_BM, d), lambda i: (i, 0)),
        out_shape=jax.ShapeDtypeStruct((NPAD, d), jnp.float32),
    )(h, w_, b_.reshape(1, d))


def _out_relu_body(h_ref, ws_ref, hn_ref, wn_ref, b_ref, o_ref):
    acc = jnp.dot(h_ref[...], ws_ref[...],
                  preferred_element_type=jnp.float32,
                  precision=lax.Precision.HIGHEST)
    acc += jnp.dot(hn_ref[...], wn_ref[...],
                   preferred_element_type=jnp.float32,
                   precision=lax.Precision.HIGHEST)
    o_ref[...] = jnp.maximum(acc + b_ref[...], 0.0)


def _out_relu(h, ws, hn, wn, b_):
    k, d = ws.shape
    return pl.pallas_call(
        _out_relu_body,
        grid=(NPAD // _BM,),
        in_specs=[
            pl.BlockSpec((_BM, k), lambda i: (i, 0)),
            pl.BlockSpec((k, d), lambda i: (0, 0)),
            pl.BlockSpec((_BM, k), lambda i: (i, 0)),
            pl.BlockSpec((k, d), lambda i: (0, 0)),
            pl.BlockSpec((1, d), lambda i: (0, 0)),
        ],
        out_specs=pl.BlockSpec((_BM, d), lambda i: (i, 0)),
        out_shape=jax.ShapeDtypeStruct((NPAD, d), jnp.float32),
    )(h, ws, hn, wn, b_.reshape(1, d))


def _pad2(a, r, c):
    return jnp.pad(a, ((0, r - a.shape[0]), (0, c - a.shape[1])))


def _pad1(a, c):
    return jnp.pad(a, ((0, c - a.shape[0]),))


def kernel(x, edge_index,
           Wp0, bp0, Ws0, Wn0, b0,
           Wp1, bp1, Ws1, Wn1, b1,
           Wp2, bp2, Ws2, Wn2, b2,
           Wp3, bp3, Ws3, Wn3, b3):
    src = edge_index[0].astype(jnp.int32)
    dst = edge_index[1].astype(jnp.int32)
    srcl, dstl, cnt = _prep(src, dst)

    h = jnp.pad(x, ((0, NPAD - N), (0, 0)))
    params = [(Wp0, bp0, Ws0, Wn0, b0), (Wp1, bp1, Ws1, Wn1, b1),
              (Wp2, bp2, Ws2, Wn2, b2), (Wp3, bp3, Ws3, Wn3, b3)]
    for (Wp, bp, Ws, Wn, b) in params:
        p = _linear_relu(h, _pad2(Wp, D, D), _pad1(bp, D))
        hn = _segmax(p, srcl, dstl, cnt)
        h = _out_relu(h, _pad2(Ws, D, D), hn, _pad2(Wn, D, D), _pad1(b, D))
    return h[:N]
